# baseline (speedup 1.0000x reference)
import functools

import jax
import jax.numpy as jnp
from jax import lax
from jax.experimental import pallas as pl
from jax.experimental.pallas import tpu as pltpu
from jax.experimental.pallas import tpu_sc as plsc

DIM = 1024
HEADS = 16
DH = 64
GS = 128
WIN = 2 * GS
NB = 2
N = 4096
NG = N // GS
MG = 128
SCALE = DH ** -0.5


def _sc_gather(table, idx, chunk):
    R, D = table.shape
    (M,) = idx.shape
    info = plsc.get_sparse_core_info()
    nw = info.num_cores * info.num_subcores
    per_w = M // nw
    assert per_w * nw == M and per_w % chunk == 0 and chunk % 8 == 0
    nchunks = per_w // chunk
    mesh = plsc.VectorSubcoreMesh(core_axis_name="c", subcore_axis_name="s")

    @functools.partial(
        pl.kernel,
        mesh=mesh,
        out_type=jax.ShapeDtypeStruct((M, D), table.dtype),
        scratch_types=[
            pltpu.VMEM((per_w,), jnp.int32),
            pltpu.VMEM((chunk, D), table.dtype),
            pltpu.VMEM((chunk, D), table.dtype),
            pltpu.SemaphoreType.DMA,
            pltpu.SemaphoreType.DMA,
        ],
    )
    def gk(table_hbm, idx_hbm, out_hbm, idx_v, rows_a, rows_b, sem_a, sem_b):
        wid = lax.axis_index("s") * info.num_cores + lax.axis_index("c")
        base = wid * per_w
        pltpu.sync_copy(idx_hbm.at[pl.ds(base, per_w)], idx_v)
        bufs = (rows_a, rows_b)
        sems = (sem_a, sem_b)
        copies = [
            pltpu.async_copy(
                table_hbm.at[idx_v.at[pl.ds(c * chunk, chunk)]],
                bufs[c % 2],
                sems[c % 2],
            )
            for c in range(min(2, nchunks))
        ]
        for c in range(nchunks):
            copies[c].wait()
            pltpu.sync_copy(bufs[c % 2], out_hbm.at[pl.ds(base + c * chunk, chunk)])
            if c + 2 < nchunks:
                copies.append(
                    pltpu.async_copy(
                        table_hbm.at[idx_v.at[pl.ds((c + 2) * chunk, chunk)]],
                        bufs[c % 2],
                        sems[c % 2],
                    )
                )

    return gk(table, idx)


def _fused(xg, k_global, v_global, wq, wk, wv, wp):
    nb = xg.shape[1] // GS
    c = SCALE * 1.4426950408889634
    bf = jnp.bfloat16
    f32 = jnp.float32
    dn = (((1,), (1,)), ((), ()))

    def body(x_ref, kg_ref, vg_ref, wq_ref, wk_ref, wv_ref, wp_ref, ones_ref,
             probs_ref, y_ref, kr, vr, qr, acc_ref):
        g = pl.program_id(1)
        slot = lax.rem(g, 2)
        x = x_ref[0].astype(bf)
        kr[slot] = lax.dot_general(x, wk_ref[...], dn,
                                   preferred_element_type=f32).astype(bf)
        vr[slot] = lax.dot_general(x, wv_ref[...], dn,
                                   preferred_element_type=f32).astype(bf)
        qr[slot] = lax.dot_general(x, wq_ref[...], dn,
                                   preferred_element_type=f32).astype(bf)

        @pl.when(g > 0)
        def _attn_part():
            prev = 1 - slot
            ones = ones_ref[...]

            def head_logits(h):
                sl = slice(h * DH, (h + 1) * DH)
                qh = qr[prev, :, sl]
                la = lax.dot_general(qh, kr[prev, :, sl], dn,
                                     preferred_element_type=f32)
                lb = lax.dot_general(qh, kr[slot, :, sl], dn,
                                     preferred_element_type=f32)
                gl = lax.dot_general(qh, kg_ref[h], dn,
                                     preferred_element_type=f32)
                return la, lb, gl

            def head_post(h, la, lb, gl):
                sl = slice(h * DH, (h + 1) * DH)
                ea = jnp.exp2(la * c)
                eb = jnp.exp2(lb * c)
                eab = (ea + eb).astype(bf)
                s = jnp.dot(eab, ones, preferred_element_type=f32)
                r = 1.0 / s
                probs_ref[0, 0, h, :, :GS] = ea * r
                probs_ref[0, 0, h, :, GS:] = eb * r
                o1 = (jnp.dot(ea.astype(bf), vr[prev, :, sl],
                              preferred_element_type=f32)
                      + jnp.dot(eb.astype(bf), vr[slot, :, sl],
                                preferred_element_type=f32)) * r[:, :DH]
                ge = jnp.exp2(gl * c).astype(bf)
                sg = jnp.dot(ge, ones, preferred_element_type=f32)
                o2 = jnp.dot(ge, vg_ref[h],
                             preferred_element_type=f32) / sg[:, :DH]
                acc_ref[:, sl] = o1 + o2

            prev_h = None
            for h in range(HEADS):
                cur = (h,) + head_logits(h)
                if prev_h is not None:
                    head_post(*prev_h)
                prev_h = cur
            head_post(*prev_h)
            y_ref[0] = lax.dot_general(acc_ref[...].astype(bf), wp_ref[...],
                                       dn, preferred_element_type=f32)

    back = lambda b, g: (b, jnp.maximum(g - 1, 0), 0)
    return pl.pallas_call(
        body,
        grid=(NB, nb),
        in_specs=[
            pl.BlockSpec((1, GS, DIM), lambda b, g: (b, g, 0)),
            pl.BlockSpec((HEADS, MG, DH), lambda b, g: (0, 0, 0)),
            pl.BlockSpec((HEADS, MG, DH), lambda b, g: (0, 0, 0)),
            pl.BlockSpec((DIM, DIM), lambda b, g: (0, 0)),
            pl.BlockSpec((DIM, DIM), lambda b, g: (0, 0)),
            pl.BlockSpec((DIM, DIM), lambda b, g: (0, 0)),
            pl.BlockSpec((DIM, DIM), lambda b, g: (0, 0)),
            pl.BlockSpec((GS, GS), lambda b, g: (0, 0)),
        ],
        out_specs=[
            pl.BlockSpec((1, 1, HEADS, GS, WIN),
                         lambda b, g: (b, jnp.maximum(g - 1, 0), 0, 0, 0)),
            pl.BlockSpec((1, GS, DIM), back),
        ],
        out_shape=[
            jax.ShapeDtypeStruct((NB, NG, HEADS, GS, WIN), jnp.float32),
            jax.ShapeDtypeStruct((NB, N, DIM), jnp.float32),
        ],
        scratch_shapes=[
            pltpu.VMEM((2, GS, DIM), jnp.bfloat16),
            pltpu.VMEM((2, GS, DIM), jnp.bfloat16),
            pltpu.VMEM((2, GS, DIM), jnp.bfloat16),
            pltpu.VMEM((GS, DIM), jnp.float32),
        ],
    )(xg, k_global, v_global, wq, wk, wv, wp,
      jnp.ones((GS, GS), jnp.bfloat16))


def kernel(normed_x, idx_last, k_global, v_global, Wq, Wk, Wv, Wproj):
    b, n, d = normed_x.shape
    idx = idx_last[..., 0].astype(jnp.int32)
    idx_ext = jnp.concatenate([idx, idx[:, n - GS:][:, ::-1]], axis=1)
    boff = (jnp.arange(b, dtype=jnp.int32) * n)[:, None]
    flat_idx = (idx_ext + boff).reshape(-1)
    xg = _sc_gather(normed_x.reshape(b * n, d), flat_idx, chunk=24)
    xg = xg.reshape(b, n + GS, d)

    bf = jnp.bfloat16
    probs, y = _fused(xg, k_global.astype(bf), v_global.astype(bf),
                      Wq.astype(bf), Wk.astype(bf), Wv.astype(bf),
                      Wproj.astype(bf))

    idx_d = idx + (xg[0, 0, 0] * 0.0).astype(jnp.int32)
    ar = jnp.arange(n, dtype=jnp.int32)
    maxsrc = jax.vmap(
        lambda i: jnp.full((n,), -1, jnp.int32).at[i].max(ar))(idx_d)
    src = jnp.where(maxsrc >= 0, maxsrc, ar[None])
    src_flat = (src + boff).reshape(-1)
    out = _sc_gather(y.reshape(b * n, d), src_flat, chunk=32).reshape(b, n, d)
    return out, probs
